# baseline (speedup 1.0000x reference)
import functools

import jax
import jax.numpy as jnp
from jax import lax
from jax.experimental import pallas as pl
from jax.experimental.pallas import tpu as pltpu
from jax.experimental.pallas import tpu_sc as plsc

VOCAB = 1000000
EMB = 64
HID = 256
OUT = 174
B = 4096
L = 200

NC = 2
NS = 16
NW = NC * NS
BPW = B // NW
NCH = 2
CH = L // NCH


def _sc_pool(x3, table):
    mesh = plsc.VectorSubcoreMesh(core_axis_name="c", subcore_axis_name="s")

    @functools.partial(
        pl.kernel,
        mesh=mesh,
        out_type=jax.ShapeDtypeStruct((B, EMB), jnp.float32),
        compiler_params=pltpu.CompilerParams(use_tc_tiling_on_sc=False),
        scratch_types=[
            pltpu.VMEM((BPW, NCH, CH), jnp.int32),
            pltpu.VMEM((NCH, CH, EMB), jnp.float32),
            pltpu.VMEM((BPW, EMB), jnp.float32),
            pltpu.SemaphoreType.DMA,
        ],
    )
    def k(x_hbm, tab_hbm, out_hbm, idx_v, rows_v, acc_v, sem):
        wid = lax.axis_index("s") * NC + lax.axis_index("c")
        pltpu.sync_copy(x_hbm.at[wid], idx_v)

        def body(b, carry):
            cps = [
                pltpu.async_copy(tab_hbm.at[idx_v.at[b, j]], rows_v.at[j], sem)
                for j in range(NCH)
            ]
            for cp in cps:
                cp.wait()
            z = jnp.zeros((16,), jnp.float32)
            accs = (z, z, z, z)
            for j in range(NCH):
                def red(r, a, j=j):
                    return tuple(
                        a[c] + rows_v[j, r, pl.ds(c * 16, 16)] for c in range(4)
                    )
                accs = lax.fori_loop(0, CH, red, accs)
            for c in range(4):
                acc_v[b, pl.ds(c * 16, 16)] = accs[c]
            return carry

        lax.fori_loop(0, BPW, body, 0)
        pltpu.sync_copy(acc_v, out_hbm.at[pl.ds(wid * BPW, BPW)])

    return k(x3, table)


def _mlp_body(p_ref, w1_ref, b1_ref, w2_ref, b2_ref, o_ref):
    p = p_ref[...] * (1.0 / L)
    h = jnp.maximum(
        jnp.dot(p, w1_ref[...], preferred_element_type=jnp.float32) + b1_ref[...],
        0.0,
    )
    z = jnp.dot(h, w2_ref[...], preferred_element_type=jnp.float32) + b2_ref[...]
    o_ref[...] = 1.0 / (1.0 + jnp.exp(-z))


def _mlp(pooled, W1, b1, W2, b2):
    BLK = 512
    grid = (B // BLK,)
    return pl.pallas_call(
        _mlp_body,
        grid=grid,
        in_specs=[
            pl.BlockSpec((BLK, EMB), lambda i: (i, 0)),
            pl.BlockSpec((EMB, HID), lambda i: (0, 0)),
            pl.BlockSpec((1, HID), lambda i: (0, 0)),
            pl.BlockSpec((HID, OUT), lambda i: (0, 0)),
            pl.BlockSpec((1, OUT), lambda i: (0, 0)),
        ],
        out_specs=pl.BlockSpec((BLK, OUT), lambda i: (i, 0)),
        out_shape=jax.ShapeDtypeStruct((B, OUT), jnp.float32),
    )(pooled, W1, b1.reshape(1, HID), W2, b2.reshape(1, OUT))


def kernel(x, table, W1, b1, W2, b2):
    x3 = x.astype(jnp.int32).reshape(NW, BPW, NCH, CH)
    pooled = _sc_pool(x3, table)
    return _mlp(pooled, W1, b1, W2, b2)


# speedup vs baseline: 1.2420x; 1.2420x over previous
import functools

import jax
import jax.numpy as jnp
from jax import lax
from jax.experimental import pallas as pl
from jax.experimental.pallas import tpu as pltpu
from jax.experimental.pallas import tpu_sc as plsc

VOCAB = 1000000
EMB = 64
HID = 256
OUT = 174
B = 4096
L = 200

NC = 2
NS = 16
NW = NC * NS
BPW = B // NW
NCH = 2
CH = L // NCH
NBUF = 4
RU = 4


def _sc_pool(x3, table):
    mesh = plsc.VectorSubcoreMesh(core_axis_name="c", subcore_axis_name="s")

    @functools.partial(
        pl.kernel,
        mesh=mesh,
        out_type=jax.ShapeDtypeStruct((B, EMB), jnp.float32),
        compiler_params=pltpu.CompilerParams(use_tc_tiling_on_sc=False),
        scratch_types=[
            pltpu.VMEM((BPW, NCH, CH), jnp.int32),
            pltpu.VMEM((NBUF, NCH, CH, EMB), jnp.float32),
            pltpu.VMEM((BPW, EMB), jnp.float32),
            [pltpu.SemaphoreType.DMA] * NBUF,
        ],
    )
    def k(x_hbm, tab_hbm, out_hbm, idx_v, rows_v, acc_v, sems):
        wid = lax.axis_index("s") * NC + lax.axis_index("c")
        pltpu.sync_copy(x_hbm.at[wid], idx_v)

        def fire(b, s):
            for j in range(NCH):
                pltpu.async_copy(
                    tab_hbm.at[idx_v.at[b, j]], rows_v.at[s, j], sems[s]
                )

        def wait(b, s):
            for j in range(NCH):
                pltpu.make_async_copy(
                    tab_hbm.at[idx_v.at[b, j]], rows_v.at[s, j], sems[s]
                ).wait()

        def reduce(b, s):
            z = jnp.zeros((16,), jnp.float32)

            def red(r4, a):
                for j in range(NCH):
                    for dr in range(RU):
                        r = r4 * RU + dr
                        a = tuple(
                            a[c] + rows_v[s, j, r, pl.ds(c * 16, 16)]
                            for c in range(4)
                        )
                return a

            accs = lax.fori_loop(0, CH // RU, red, (z, z, z, z))
            for c in range(4):
                acc_v[b, pl.ds(c * 16, 16)] = accs[c]

        for s in range(NBUF):
            fire(s, s)

        def body(g, carry):
            for s in range(NBUF):
                b = g * NBUF + s
                wait(b, s)
                bn = b + NBUF

                @pl.when(bn < BPW)
                def _():
                    fire(bn, s)

                reduce(b, s)
            return carry

        lax.fori_loop(0, BPW // NBUF, body, 0)
        pltpu.sync_copy(acc_v, out_hbm.at[pl.ds(wid * BPW, BPW)])

    return k(x3, table)


def _mlp_body(p_ref, w1_ref, b1_ref, w2_ref, b2_ref, o_ref):
    p = p_ref[...] * (1.0 / L)
    h = jnp.maximum(
        jnp.dot(p, w1_ref[...], preferred_element_type=jnp.float32) + b1_ref[...],
        0.0,
    )
    z = jnp.dot(h, w2_ref[...], preferred_element_type=jnp.float32) + b2_ref[...]
    o_ref[...] = 1.0 / (1.0 + jnp.exp(-z))


def _mlp(pooled, W1, b1, W2, b2):
    BLK = 512
    grid = (B // BLK,)
    return pl.pallas_call(
        _mlp_body,
        grid=grid,
        in_specs=[
            pl.BlockSpec((BLK, EMB), lambda i: (i, 0)),
            pl.BlockSpec((EMB, HID), lambda i: (0, 0)),
            pl.BlockSpec((1, HID), lambda i: (0, 0)),
            pl.BlockSpec((HID, OUT), lambda i: (0, 0)),
            pl.BlockSpec((1, OUT), lambda i: (0, 0)),
        ],
        out_specs=pl.BlockSpec((BLK, OUT), lambda i: (i, 0)),
        out_shape=jax.ShapeDtypeStruct((B, OUT), jnp.float32),
    )(pooled, W1, b1.reshape(1, HID), W2, b2.reshape(1, OUT))


def kernel(x, table, W1, b1, W2, b2):
    x3 = x.astype(jnp.int32).reshape(NW, BPW, NCH, CH)
    pooled = _sc_pool(x3, table)
    return _mlp(pooled, W1, b1, W2, b2)


# speedup vs baseline: 1.3685x; 1.1019x over previous
import functools

import jax
import jax.numpy as jnp
from jax import lax
from jax.experimental import pallas as pl
from jax.experimental.pallas import tpu as pltpu
from jax.experimental.pallas import tpu_sc as plsc

VOCAB = 1000000
EMB = 64
HID = 256
OUT = 174
B = 4096
L = 200

NC = 2
NS = 16
NW = NC * NS
BPW = B // NW
NCH = 2
CH = L // NCH
NBUF = 4
RU = 4


def _sc_pool(x3, table):
    mesh = plsc.VectorSubcoreMesh(core_axis_name="c", subcore_axis_name="s")

    @functools.partial(
        pl.kernel,
        mesh=mesh,
        out_type=jax.ShapeDtypeStruct((B, EMB), jnp.float32),
        compiler_params=pltpu.CompilerParams(use_tc_tiling_on_sc=False),
        scratch_types=[
            pltpu.VMEM((BPW, NCH, CH), jnp.int32),
            pltpu.VMEM((NBUF, NCH, CH, EMB), jnp.float32),
            pltpu.VMEM((BPW, EMB), jnp.float32),
            [pltpu.SemaphoreType.DMA] * NBUF,
        ],
    )
    def k(x_hbm, tab_hbm, out_hbm, idx_v, rows_v, acc_v, sems):
        wid = lax.axis_index("s") * NC + lax.axis_index("c")
        pltpu.sync_copy(x_hbm.at[wid], idx_v)

        def fire(b, s):
            for j in range(NCH):
                pltpu.async_copy(
                    tab_hbm.at[idx_v.at[b, j]], rows_v.at[s, j], sems[s]
                )

        def wait(b, s):
            for j in range(NCH):
                pltpu.make_async_copy(
                    tab_hbm.at[idx_v.at[b, j]], rows_v.at[s, j], sems[s]
                ).wait()

        def reduce(b, s):
            z = jnp.zeros((16,), jnp.float32)

            def red(r4, a):
                for j in range(NCH):
                    for dr in range(RU):
                        r = r4 * RU + dr
                        a = tuple(
                            a[c] + rows_v[s, j, r, pl.ds(c * 16, 16)]
                            for c in range(4)
                        )
                return a

            accs = lax.fori_loop(0, CH // RU, red, (z, z, z, z))
            for c in range(4):
                acc_v[b, pl.ds(c * 16, 16)] = accs[c]

        for s in range(NBUF):
            fire(s, s)

        def body(g, carry):
            for s in range(NBUF):
                b = g * NBUF + s
                wait(b, s)
                bn = b + NBUF

                @pl.when(bn < BPW)
                def _():
                    fire(bn, s)

                reduce(b, s)
            return carry

        lax.fori_loop(0, BPW // NBUF, body, 0)
        pltpu.sync_copy(acc_v, out_hbm.at[pl.ds(wid * BPW, BPW)])

    return k(x3, table)


def _mlp_body(p_ref, w1_ref, b1_ref, w2_ref, b2_ref, o_ref):
    p = p_ref[...] * (1.0 / L)
    h = jnp.maximum(
        jnp.dot(p, w1_ref[...], preferred_element_type=jnp.float32) + b1_ref[...],
        0.0,
    )
    z = jnp.dot(h, w2_ref[...], preferred_element_type=jnp.float32) + b2_ref[...]
    o_ref[...] = 1.0 / (1.0 + jnp.exp(-z))


def _mlp(pooled, W1, b1, W2, b2):
    BLK = 512
    grid = (B // BLK,)
    return pl.pallas_call(
        _mlp_body,
        grid=grid,
        in_specs=[
            pl.BlockSpec((BLK, EMB), lambda i: (i, 0)),
            pl.BlockSpec((EMB, HID), lambda i: (0, 0)),
            pl.BlockSpec((1, HID), lambda i: (0, 0)),
            pl.BlockSpec((HID, OUT), lambda i: (0, 0)),
            pl.BlockSpec((1, OUT), lambda i: (0, 0)),
        ],
        out_specs=pl.BlockSpec((BLK, OUT), lambda i: (i, 0)),
        out_shape=jax.ShapeDtypeStruct((B, OUT), jnp.float32),
    )(pooled, W1, b1.reshape(1, HID), W2, b2.reshape(1, OUT))


def kernel(x, table, W1, b1, W2, b2):
    tpad = jnp.pad(table, ((0, 0), (0, EMB)))
    tv = tpad.reshape(2 * VOCAB, EMB)
    x2 = (x.astype(jnp.int32) * 2).reshape(NW, BPW, NCH, CH)
    pooled = _sc_pool(x2, tv)
    return _mlp(pooled, W1, b1, W2, b2)


# speedup vs baseline: 1.5044x; 1.0993x over previous
import functools

import jax
import jax.numpy as jnp
from jax import lax
from jax.experimental import pallas as pl
from jax.experimental.pallas import tpu as pltpu
from jax.experimental.pallas import tpu_sc as plsc

VOCAB = 1000000
EMB = 64
HID = 256
OUT = 174
B = 4096
L = 200

NC = 2
NS = 16
NW = NC * NS
BPW = B // NW
NCH = 2
CH = L // NCH
NBUF = 4
RU = 4


def _sc_pool(x3, table):
    mesh = plsc.VectorSubcoreMesh(core_axis_name="c", subcore_axis_name="s")

    @functools.partial(
        pl.kernel,
        mesh=mesh,
        out_type=jax.ShapeDtypeStruct((B, EMB), jnp.float32),
        compiler_params=pltpu.CompilerParams(use_tc_tiling_on_sc=False),
        scratch_types=[
            pltpu.VMEM((BPW, NCH, CH), jnp.int32),
            pltpu.VMEM((NBUF, NCH, CH, EMB), jnp.float32),
            pltpu.VMEM((BPW, EMB), jnp.float32),
            [pltpu.SemaphoreType.DMA] * NBUF,
        ],
    )
    def k(x_hbm, tab_hbm, out_hbm, idx_v, rows_v, acc_v, sems):
        wid = lax.axis_index("s") * NC + lax.axis_index("c")
        pltpu.sync_copy(x_hbm.at[wid], idx_v)

        def fire(b, s):
            for j in range(NCH):
                pltpu.async_copy(
                    tab_hbm.at[idx_v.at[b, j]], rows_v.at[s, j], sems[s]
                )

        def wait(b, s):
            for j in range(NCH):
                pltpu.make_async_copy(
                    tab_hbm.at[idx_v.at[b, j]], rows_v.at[s, j], sems[s]
                ).wait()

        def reduce(b, s):
            z = jnp.zeros((16,), jnp.float32)

            def red(r4, a):
                for j in range(NCH):
                    for dr in range(RU):
                        r = r4 * RU + dr
                        a = tuple(
                            a[c] + rows_v[s, j, r, pl.ds(c * 16, 16)]
                            for c in range(4)
                        )
                return a

            accs = lax.fori_loop(0, CH // RU, red, (z, z, z, z))
            for c in range(4):
                acc_v[b, pl.ds(c * 16, 16)] = accs[c]

        for s in range(NBUF):
            fire(s, s)

        def body(g, carry):
            for s in range(NBUF):
                b = g * NBUF + s
                wait(b, s)
                bn = b + NBUF

                @pl.when(bn < BPW)
                def _():
                    fire(bn, s)

                reduce(b, s)
            return carry

        lax.fori_loop(0, BPW // NBUF, body, 0)
        pltpu.sync_copy(acc_v, out_hbm.at[pl.ds(wid * BPW, BPW)])

    return k(x3, table)


TVB = 2048


def _tp_body(in_ref, o_ref):
    a = in_ref[...]
    at = jnp.transpose(a)
    o_ref[:, :EMB] = at[: TVB // 2]
    o_ref[:, EMB:] = at[TVB // 2 :]


def _transpose_pack(tt):
    return pl.pallas_call(
        _tp_body,
        grid=(VOCAB // TVB,),
        in_specs=[pl.BlockSpec((EMB, TVB), lambda i: (0, i))],
        out_specs=pl.BlockSpec((TVB // 2, 2 * EMB), lambda i: (i, 0)),
        out_shape=jax.ShapeDtypeStruct((VOCAB // 2, 2 * EMB), jnp.float32),
    )(tt)


def _mlp_body(p_ref, w1_ref, b1_ref, w2_ref, b2_ref, o_ref):
    p = p_ref[...] * (1.0 / L)
    h = jnp.maximum(
        jnp.dot(p, w1_ref[...], preferred_element_type=jnp.float32) + b1_ref[...],
        0.0,
    )
    z = jnp.dot(h, w2_ref[...], preferred_element_type=jnp.float32) + b2_ref[...]
    o_ref[...] = 1.0 / (1.0 + jnp.exp(-z))


def _mlp(pooled, W1, b1, W2, b2):
    BLK = 512
    grid = (B // BLK,)
    return pl.pallas_call(
        _mlp_body,
        grid=grid,
        in_specs=[
            pl.BlockSpec((BLK, EMB), lambda i: (i, 0)),
            pl.BlockSpec((EMB, HID), lambda i: (0, 0)),
            pl.BlockSpec((1, HID), lambda i: (0, 0)),
            pl.BlockSpec((HID, OUT), lambda i: (0, 0)),
            pl.BlockSpec((1, OUT), lambda i: (0, 0)),
        ],
        out_specs=pl.BlockSpec((BLK, OUT), lambda i: (i, 0)),
        out_shape=jax.ShapeDtypeStruct((B, OUT), jnp.float32),
    )(pooled, W1, b1.reshape(1, HID), W2, b2.reshape(1, OUT))


def kernel(x, table, W1, b1, W2, b2):
    tp = _transpose_pack(table.T)
    tv = tp.reshape(VOCAB, EMB)
    xi = x.astype(jnp.int32)
    r = xi & (TVB - 1)
    x2 = (xi - r + jnp.where(r < TVB // 2, 2 * r, 2 * r - TVB + 1)).reshape(
        NW, BPW, NCH, CH
    )
    pooled = _sc_pool(x2, tv)
    return _mlp(pooled, W1, b1, W2, b2)


# speedup vs baseline: 2.2609x; 1.5029x over previous
import functools

import jax
import jax.numpy as jnp
from jax import lax
from jax.experimental import pallas as pl
from jax.experimental.pallas import tpu as pltpu
from jax.experimental.pallas import tpu_sc as plsc

VOCAB = 1000000
EMB = 64
HID = 256
OUT = 174
B = 4096
L = 200

NC = 2
NS = 16
NW = NC * NS
BPW = B // NW
NCH = 2
CH = L // NCH
NBUF = 4
RU = 4


def _sc_pool(x3, table):
    mesh = plsc.VectorSubcoreMesh(core_axis_name="c", subcore_axis_name="s")

    @functools.partial(
        pl.kernel,
        mesh=mesh,
        out_type=jax.ShapeDtypeStruct((B, EMB), jnp.float32),
        compiler_params=pltpu.CompilerParams(use_tc_tiling_on_sc=False),
        scratch_types=[
            pltpu.VMEM((BPW, NCH, CH), jnp.int32),
            pltpu.VMEM((NBUF, NCH, CH, EMB), jnp.float32),
            pltpu.VMEM((BPW, EMB), jnp.float32),
            [pltpu.SemaphoreType.DMA] * NBUF,
        ],
    )
    def k(x_hbm, tab_hbm, out_hbm, idx_v, rows_v, acc_v, sems):
        wid = lax.axis_index("s") * NC + lax.axis_index("c")
        pltpu.sync_copy(x_hbm.at[wid], idx_v)

        def fire(b, s):
            for j in range(NCH):
                pltpu.async_copy(
                    tab_hbm.at[idx_v.at[b, j]], rows_v.at[s, j], sems[s]
                )

        def wait(b, s):
            for j in range(NCH):
                pltpu.make_async_copy(
                    tab_hbm.at[idx_v.at[b, j]], rows_v.at[s, j], sems[s]
                ).wait()

        def reduce(b, s):
            z = jnp.zeros((16,), jnp.float32)

            def red(r4, a):
                for j in range(NCH):
                    for dr in range(RU):
                        r = r4 * RU + dr
                        a = tuple(
                            a[c] + rows_v[s, j, r, pl.ds(c * 16, 16)]
                            for c in range(4)
                        )
                return a

            accs = lax.fori_loop(0, CH // RU, red, (z, z, z, z))
            for c in range(4):
                acc_v[b, pl.ds(c * 16, 16)] = accs[c]

        for s in range(NBUF):
            fire(s, s)

        def body(g, carry):
            for s in range(NBUF):
                b = g * NBUF + s
                wait(b, s)
                bn = b + NBUF

                @pl.when(bn < BPW)
                def _():
                    fire(bn, s)

                reduce(b, s)
            return carry

        lax.fori_loop(0, BPW // NBUF, body, 0)
        pltpu.sync_copy(acc_v, out_hbm.at[pl.ds(wid * BPW, BPW)])

    return k(x3, table)


TVB = 8192


def _tp_body(in_ref, o_ref):
    a = in_ref[...]
    at = jnp.transpose(a)
    o_ref[:, :EMB] = at[: TVB // 2]
    o_ref[:, EMB:] = at[TVB // 2 :]


def _transpose_pack(tt):
    return pl.pallas_call(
        _tp_body,
        grid=(VOCAB // TVB,),
        in_specs=[pl.BlockSpec((EMB, TVB), lambda i: (0, i))],
        out_specs=pl.BlockSpec((TVB // 2, 2 * EMB), lambda i: (i, 0)),
        out_shape=jax.ShapeDtypeStruct((VOCAB // 2, 2 * EMB), jnp.float32),
    )(tt)


def _mlp_body(p_ref, w1_ref, b1_ref, w2_ref, b2_ref, o_ref):
    p = p_ref[...] * (1.0 / L)
    h = jnp.maximum(
        jnp.dot(p, w1_ref[...], preferred_element_type=jnp.float32) + b1_ref[...],
        0.0,
    )
    z = jnp.dot(h, w2_ref[...], preferred_element_type=jnp.float32) + b2_ref[...]
    o_ref[...] = 1.0 / (1.0 + jnp.exp(-z))


def _mlp(pooled, W1, b1, W2, b2):
    BLK = 512
    grid = (B // BLK,)
    return pl.pallas_call(
        _mlp_body,
        grid=grid,
        in_specs=[
            pl.BlockSpec((BLK, EMB), lambda i: (i, 0)),
            pl.BlockSpec((EMB, HID), lambda i: (0, 0)),
            pl.BlockSpec((1, HID), lambda i: (0, 0)),
            pl.BlockSpec((HID, OUT), lambda i: (0, 0)),
            pl.BlockSpec((1, OUT), lambda i: (0, 0)),
        ],
        out_specs=pl.BlockSpec((BLK, OUT), lambda i: (i, 0)),
        out_shape=jax.ShapeDtypeStruct((B, OUT), jnp.float32),
    )(pooled, W1, b1.reshape(1, HID), W2, b2.reshape(1, OUT))


def kernel(x, table, W1, b1, W2, b2):
    tp = _transpose_pack(table.T)
    tv = tp.reshape(VOCAB, EMB)
    xi = x.astype(jnp.int32)
    r = xi & (TVB - 1)
    x2 = (xi - r + jnp.where(r < TVB // 2, 2 * r, 2 * r - TVB + 1)).reshape(
        NW, BPW, NCH, CH
    )
    pooled = _sc_pool(x2, tv)
    return _mlp(pooled, W1, b1, W2, b2)
